# baseline (speedup 1.0000x reference)
import functools
import math

import jax
import jax.numpy as jnp
from jax import lax
from jax.experimental import pallas as pl
from jax.experimental.pallas import tpu as pltpu
from jax.experimental.pallas import tpu_sc as plsc

_NUM_CORES = 2
_NUM_SUBCORES = 16
_NUM_WORKERS = _NUM_CORES * _NUM_SUBCORES
_LANES = 16
_GROUP = 128


def _make_emb_kernel(n_rows, vocab, d):
    rows_per_w = n_rows // _NUM_WORKERS
    groups_per_w = rows_per_w // _GROUP
    g_per_chunk = 4
    chunk = g_per_chunk * _GROUP
    n_chunks = groups_per_w // g_per_chunk
    scale = math.sqrt(d)

    mesh = plsc.VectorSubcoreMesh(core_axis_name="c", subcore_axis_name="s")

    @functools.partial(
        pl.kernel,
        out_type=jax.ShapeDtypeStruct((n_rows, d), jnp.float32),
        mesh=mesh,
        compiler_params=pltpu.CompilerParams(use_tc_tiling_on_sc=False),
        scratch_types=[
            pltpu.VMEM((groups_per_w, _GROUP), jnp.int32),
            pltpu.VMEM((chunk, d), jnp.float32),
            pltpu.SemaphoreType.DMA,
        ],
    )
    def emb(table_hbm, idx_hbm, out_hbm, idx_v, buf, gsem):
        wid = lax.axis_index("s") * _NUM_CORES + lax.axis_index("c")
        grp0 = wid * groups_per_w
        row0 = wid * rows_per_w
        pltpu.sync_copy(idx_hbm.at[pl.ds(grp0, groups_per_w)], idx_v)

        def do_chunk(ch, carry):
            copies = [
                pltpu.async_copy(
                    table_hbm.at[idx_v.at[ch * g_per_chunk + g]],
                    buf.at[pl.ds(g * _GROUP, _GROUP)],
                    gsem,
                )
                for g in range(g_per_chunk)
            ]
            for c in copies:
                c.wait()

            def scale_row(r, carry2):
                for c4 in range(d // _LANES):
                    sl = pl.ds(c4 * _LANES, _LANES)
                    buf[r, sl] = buf[r, sl] * scale
                return carry2

            lax.fori_loop(0, chunk, scale_row, 0, unroll=4)
            pltpu.sync_copy(buf, out_hbm.at[pl.ds(row0 + ch * chunk, chunk)])
            return carry

        lax.fori_loop(0, n_chunks, do_chunk, 0)

    return emb


def kernel(x, table):
    b, s = x.shape
    vocab, d = table.shape
    n_rows = b * s
    xf = x.astype(jnp.int32).reshape(n_rows // _GROUP, _GROUP)
    emb = _make_emb_kernel(n_rows, vocab, d)
    out = emb(table, xf)
    return out.reshape(b, s, d)
